# speedup vs baseline: 1.3461x; 1.3461x over previous
import functools

import jax
import jax.numpy as jnp
from jax import lax
from jax.experimental import pallas as pl
from jax.experimental.pallas import tpu as pltpu
from jax.experimental.pallas import tpu_sc as plsc

_NC = 2
_NS = 16
_NW = _NC * _NS
_LANES = 16


def _sc_gather_mean(x_flat, table, B, L, D):
    b_per_w = B // _NW
    n_idx = b_per_w * L
    n_col = D // _LANES
    chunks = []
    off = 0
    while off < L:
        c = min(128, L - off)
        chunks.append((off, c))
        off += c

    mesh = plsc.VectorSubcoreMesh(
        core_axis_name="c", subcore_axis_name="s",
        num_cores=_NC, num_subcores=_NS)

    @functools.partial(
        pl.kernel,
        out_type=jax.ShapeDtypeStruct((B, D), jnp.float32),
        mesh=mesh,
        scratch_types=[
            pltpu.VMEM((n_idx,), jnp.int32),
            pltpu.VMEM((2, L, D), jnp.float32),
            pltpu.VMEM((b_per_w, D), jnp.float32),
            pltpu.SemaphoreType.DMA,
            pltpu.SemaphoreType.DMA,
        ],
    )
    def k(x_hbm, table_hbm, h_hbm, idx_v, rows_v, h_v, sem0, sem1):
        wid = lax.axis_index("s") * _NC + lax.axis_index("c")
        base = wid * n_idx
        pltpu.sync_copy(x_hbm.at[pl.ds(base, n_idx)], idx_v)
        sems = (sem0, sem1)
        handles = [None, None]

        def start(r, buf):
            row_off = r * L
            hs = []
            for (off, c) in chunks:
                hs.append(pltpu.async_copy(
                    table_hbm.at[idx_v.at[pl.ds(row_off + off, c)]],
                    rows_v.at[buf, pl.ds(off, c)],
                    sems[buf]))
            handles[buf] = hs

        start(0, 0)
        inv = jnp.float32(1.0 / L)
        for r in range(b_per_w):
            buf = r % 2
            if r + 1 < b_per_w:
                start(r + 1, 1 - buf)
            for h in handles[buf]:
                h.wait()

            def body(j, acc, _buf=buf):
                return tuple(
                    acc[c] + rows_v[_buf, j, pl.ds(c * _LANES, _LANES)]
                    for c in range(n_col))

            acc = lax.fori_loop(
                0, L, body,
                tuple(jnp.zeros((_LANES,), jnp.float32)
                      for _ in range(n_col)))
            for c in range(n_col):
                h_v[r, pl.ds(c * _LANES, _LANES)] = acc[c] * inv
        pltpu.sync_copy(h_v, h_hbm.at[pl.ds(wid * b_per_w, b_per_w), :])

    return k(x_flat, table)


def _tc_mlp(h, W1, b1, W2, b2, block_v=1024):
    B, D = h.shape
    HID = W1.shape[1]
    V = W2.shape[1]
    grid = (pl.cdiv(V, block_v),)

    def body(h_ref, w1_ref, b1_ref, w2_ref, b2_ref, out_ref, hid_ref):
        @pl.when(pl.program_id(0) == 0)
        def _():
            hid_ref[...] = jnp.maximum(
                jnp.dot(h_ref[...], w1_ref[...],
                        preferred_element_type=jnp.float32) + b1_ref[...],
                0.0)
        out_ref[...] = jnp.dot(
            hid_ref[...], w2_ref[...],
            preferred_element_type=jnp.float32) + b2_ref[...]

    return pl.pallas_call(
        body,
        grid=grid,
        in_specs=[
            pl.BlockSpec((B, D), lambda j: (0, 0)),
            pl.BlockSpec((D, HID), lambda j: (0, 0)),
            pl.BlockSpec((1, HID), lambda j: (0, 0)),
            pl.BlockSpec((HID, block_v), lambda j: (0, j)),
            pl.BlockSpec((1, block_v), lambda j: (0, j)),
        ],
        out_specs=pl.BlockSpec((B, block_v), lambda j: (0, j)),
        out_shape=jax.ShapeDtypeStruct((B, V), jnp.float32),
        scratch_shapes=[pltpu.VMEM((B, HID), jnp.float32)],
    )(h, W1, b1.reshape(1, -1), W2, b2.reshape(1, -1))


def kernel(x, table, W1, b1, W2, b2):
    B, L = x.shape
    _, D = table.shape
    h = _sc_gather_mean(x.reshape(B * L), table, B, L, D)
    return _tc_mlp(h, W1, b1, W2, b2)


# speedup vs baseline: 1.3847x; 1.0286x over previous
import functools

import jax
import jax.numpy as jnp
from jax import lax
from jax.experimental import pallas as pl
from jax.experimental.pallas import tpu as pltpu
from jax.experimental.pallas import tpu_sc as plsc

_NC = 2
_NS = 16
_NW = _NC * _NS
_LANES = 16


def _sc_gather_mean(x_flat, table, B, L, D):
    b_per_w = B // _NW
    n_idx = b_per_w * L
    n_col = D // _LANES
    chunks = []
    off = 0
    while off < L:
        c = min(128, L - off)
        chunks.append((off, c))
        off += c

    mesh = plsc.VectorSubcoreMesh(
        core_axis_name="c", subcore_axis_name="s",
        num_cores=_NC, num_subcores=_NS)

    @functools.partial(
        pl.kernel,
        out_type=jax.ShapeDtypeStruct((B, D), jnp.float32),
        mesh=mesh,
        scratch_types=[
            pltpu.VMEM((n_idx,), jnp.int32),
            pltpu.VMEM((2, L, D), jnp.float32),
            pltpu.VMEM((b_per_w, D), jnp.float32),
            pltpu.SemaphoreType.DMA,
            pltpu.SemaphoreType.DMA,
        ],
    )
    def k(x_hbm, table_hbm, h_hbm, idx_v, rows_v, h_v, sem0, sem1):
        wid = lax.axis_index("s") * _NC + lax.axis_index("c")
        base = wid * n_idx
        pltpu.sync_copy(x_hbm.at[pl.ds(base, n_idx)], idx_v)
        sems = (sem0, sem1)
        handles = [None, None]

        def start(r, buf):
            row_off = r * L
            hs = []
            for (off, c) in chunks:
                hs.append(pltpu.async_copy(
                    table_hbm.at[idx_v.at[pl.ds(row_off + off, c)]],
                    rows_v.at[buf, pl.ds(off, c)],
                    sems[buf]))
            handles[buf] = hs

        start(0, 0)
        inv = jnp.float32(1.0 / L)
        for r in range(b_per_w):
            buf = r % 2
            if r + 1 < b_per_w:
                start(r + 1, 1 - buf)
            for h in handles[buf]:
                h.wait()

            def body(j, acc, _buf=buf):
                return tuple(
                    acc[c] + rows_v[_buf, j, pl.ds(c * _LANES, _LANES)]
                    for c in range(n_col))

            acc = lax.fori_loop(
                0, L, body,
                tuple(jnp.zeros((_LANES,), jnp.float32)
                      for _ in range(n_col)))
            for c in range(n_col):
                h_v[r, pl.ds(c * _LANES, _LANES)] = acc[c] * inv
        pltpu.sync_copy(h_v, h_hbm.at[pl.ds(wid * b_per_w, b_per_w), :])

    return k(x_flat, table)


def _tc_mlp(h, W1, b1, W2, b2, block_v=2048):
    B, D = h.shape
    HID = W1.shape[1]
    V = W2.shape[1]
    grid = (pl.cdiv(V, block_v),)

    def body(h_ref, w1_ref, b1_ref, w2_ref, b2_ref, out_ref, hid_ref):
        @pl.when(pl.program_id(0) == 0)
        def _():
            hid_ref[...] = jnp.maximum(
                jnp.dot(h_ref[...], w1_ref[...],
                        preferred_element_type=jnp.float32) + b1_ref[...],
                0.0)
        out_ref[...] = jnp.dot(
            hid_ref[...], w2_ref[...],
            preferred_element_type=jnp.float32) + b2_ref[...]

    return pl.pallas_call(
        body,
        grid=grid,
        in_specs=[
            pl.BlockSpec((B, D), lambda j: (0, 0)),
            pl.BlockSpec((D, HID), lambda j: (0, 0)),
            pl.BlockSpec((1, HID), lambda j: (0, 0)),
            pl.BlockSpec((HID, block_v), lambda j: (0, j)),
            pl.BlockSpec((1, block_v), lambda j: (0, j)),
        ],
        out_specs=pl.BlockSpec((B, block_v), lambda j: (0, j)),
        out_shape=jax.ShapeDtypeStruct((B, V), jnp.float32),
        scratch_shapes=[pltpu.VMEM((B, HID), jnp.float32)],
    )(h, W1, b1.reshape(1, -1), W2, b2.reshape(1, -1))


def kernel(x, table, W1, b1, W2, b2):
    B, L = x.shape
    _, D = table.shape
    h = _sc_gather_mean(x.reshape(B * L), table, B, L, D)
    return _tc_mlp(h, W1, b1, W2, b2)


# speedup vs baseline: 1.3923x; 1.0055x over previous
import functools

import jax
import jax.numpy as jnp
from jax import lax
from jax.experimental import pallas as pl
from jax.experimental.pallas import tpu as pltpu
from jax.experimental.pallas import tpu_sc as plsc

_NC = 2
_NS = 16
_NW = _NC * _NS
_LANES = 16


def _sc_gather_mean(x_flat, table, B, L, D):
    b_per_w = B // _NW
    n_idx = b_per_w * L
    n_col = D // _LANES
    chunks = []
    off = 0
    while off < L:
        c = min(128, L - off)
        chunks.append((off, c))
        off += c

    mesh = plsc.VectorSubcoreMesh(
        core_axis_name="c", subcore_axis_name="s",
        num_cores=_NC, num_subcores=_NS)

    @functools.partial(
        pl.kernel,
        out_type=jax.ShapeDtypeStruct((B, D), jnp.float32),
        mesh=mesh,
        scratch_types=[
            pltpu.VMEM((n_idx,), jnp.int32),
            pltpu.VMEM((2, L, D), jnp.float32),
            pltpu.VMEM((b_per_w, D), jnp.float32),
            pltpu.SemaphoreType.DMA,
            pltpu.SemaphoreType.DMA,
        ],
    )
    def k(x_hbm, table_hbm, h_hbm, idx_v, rows_v, h_v, sem0, sem1):
        wid = lax.axis_index("s") * _NC + lax.axis_index("c")
        base = wid * n_idx
        pltpu.sync_copy(x_hbm.at[pl.ds(base, n_idx)], idx_v)
        sems = (sem0, sem1)
        handles = [None, None]

        def start(r, buf):
            row_off = r * L
            hs = []
            for (off, c) in chunks:
                hs.append(pltpu.async_copy(
                    table_hbm.at[idx_v.at[pl.ds(row_off + off, c)]],
                    rows_v.at[buf, pl.ds(off, c)],
                    sems[buf]))
            handles[buf] = hs

        start(0, 0)
        inv = jnp.float32(1.0 / L)
        for r in range(b_per_w):
            buf = r % 2
            if r + 1 < b_per_w:
                start(r + 1, 1 - buf)
            for h in handles[buf]:
                h.wait()

            def body(j, acc, _buf=buf):
                return tuple(
                    acc[c] + rows_v[_buf, j, pl.ds(c * _LANES, _LANES)]
                    for c in range(n_col))

            acc = lax.fori_loop(
                0, L, body,
                tuple(jnp.zeros((_LANES,), jnp.float32)
                      for _ in range(n_col)))
            for c in range(n_col):
                h_v[r, pl.ds(c * _LANES, _LANES)] = acc[c] * inv
        pltpu.sync_copy(h_v, h_hbm.at[pl.ds(wid * b_per_w, b_per_w), :])

    return k(x_flat, table)


def _tc_mlp(h, W1, b1, W2, b2, block_v=4096):
    B, D = h.shape
    HID = W1.shape[1]
    V = W2.shape[1]
    grid = (pl.cdiv(V, block_v),)

    def body(h_ref, w1_ref, b1_ref, w2_ref, b2_ref, out_ref, hid_ref):
        @pl.when(pl.program_id(0) == 0)
        def _():
            hid_ref[...] = jnp.maximum(
                jnp.dot(h_ref[...], w1_ref[...],
                        preferred_element_type=jnp.float32) + b1_ref[...],
                0.0)
        out_ref[...] = jnp.dot(
            hid_ref[...], w2_ref[...],
            preferred_element_type=jnp.float32) + b2_ref[...]

    return pl.pallas_call(
        body,
        grid=grid,
        in_specs=[
            pl.BlockSpec((B, D), lambda j: (0, 0)),
            pl.BlockSpec((D, HID), lambda j: (0, 0)),
            pl.BlockSpec((1, HID), lambda j: (0, 0)),
            pl.BlockSpec((HID, block_v), lambda j: (0, j)),
            pl.BlockSpec((1, block_v), lambda j: (0, j)),
        ],
        out_specs=pl.BlockSpec((B, block_v), lambda j: (0, j)),
        out_shape=jax.ShapeDtypeStruct((B, V), jnp.float32),
        scratch_shapes=[pltpu.VMEM((B, HID), jnp.float32)],
    )(h, W1, b1.reshape(1, -1), W2, b2.reshape(1, -1))


def kernel(x, table, W1, b1, W2, b2):
    B, L = x.shape
    _, D = table.shape
    h = _sc_gather_mean(x.reshape(B * L), table, B, L, D)
    return _tc_mlp(h, W1, b1, W2, b2)
